# baseline (speedup 1.0000x reference)
import dataclasses
import functools

import jax
import jax.numpy as jnp
from jax import lax
from jax.experimental import pallas as pl
from jax.experimental.pallas import tpu as pltpu
from jax.experimental.pallas import tpu_sc as plsc

_NUM_CORES = 2
_NUM_SUBCORES = 16
_NUM_WORKERS = _NUM_CORES * _NUM_SUBCORES
_CHB = 512
_TBLK = 4096


def _tc_transpose(wt):
    Dd, V = wt.shape

    def body(in_ref, out_ref):
        out_ref[...] = in_ref[...].T

    return pl.pallas_call(
        body,
        grid=(pl.cdiv(V, _TBLK),),
        in_specs=[pl.BlockSpec((Dd, _TBLK), lambda i: (0, i))],
        out_specs=pl.BlockSpec((_TBLK, Dd), lambda i: (i, 0)),
        out_shape=jax.ShapeDtypeStruct((V, Dd), wt.dtype),
    )(wt)


def kernel(token_ids, weight):
    B, S = token_ids.shape
    D = weight.shape[1]
    n_chunks = (B // _CHB) * S
    per_w = n_chunks // _NUM_WORKERS
    assert B % _CHB == 0 and n_chunks % _NUM_WORKERS == 0 and per_w % 2 == 0
    blocks_per_s = B // _CHB

    tids_t = token_ids.T
    mesh = plsc.VectorSubcoreMesh(core_axis_name="c", subcore_axis_name="s")

    @functools.partial(
        pl.kernel,
        mesh=mesh,
        out_type=jax.ShapeDtypeStruct((S, D, B), weight.dtype),
        compiler_params=dataclasses.replace(
            pltpu.CompilerParams(use_tc_tiling_on_sc=False),
            needs_layout_passes=False,
        ),
        scratch_types=[
            pltpu.VMEM((1, _CHB), jnp.int32),
            pltpu.VMEM((1, _CHB), jnp.int32),
            pltpu.VMEM((_CHB, D), jnp.float32),
            pltpu.VMEM((_CHB, D), jnp.float32),
            pltpu.VMEM((D, _CHB + 1), jnp.float32),
            pltpu.VMEM((D, _CHB + 1), jnp.float32),
            pltpu.SemaphoreType.DMA,
            pltpu.SemaphoreType.DMA,
            pltpu.SemaphoreType.DMA,
            pltpu.SemaphoreType.DMA,
        ],
    )
    def gather_kernel(
        w_hbm, idx_hbm, out_hbm,
        idx0, idx1, rows0, rows1, outt0, outt1, sg0, sg1, so0, so1,
    ):
        wid = lax.axis_index("s") * _NUM_CORES + lax.axis_index("c")
        base = wid * per_w
        idx_v = (idx0, idx1)
        rows_v = (rows0, rows1)
        outt_v = (outt0, outt1)
        sem_g = (sg0, sg1)
        sem_o = (so0, so1)
        rows16 = [lax.iota(jnp.int32, 16) + 16 * h for h in range(D // 16)]

        def chunk_pos(k):
            g = base + k
            return g // blocks_per_s, (g % blocks_per_s) * _CHB

        def launch(k, b):
            s, b0 = chunk_pos(k)
            pltpu.sync_copy(idx_hbm.at[pl.ds(s, 1), pl.ds(b0, _CHB)], idx_v[b])
            pltpu.async_copy(w_hbm.at[idx_v[b].at[0]], rows_v[b], sem_g[b])

        def drain_gather(b):
            pltpu.make_async_copy(
                w_hbm.at[idx_v[b].at[0]], rows_v[b], sem_g[b]
            ).wait()

        def drain_out(k, b):
            s, b0 = chunk_pos(k)
            pltpu.make_async_copy(
                outt_v[b].at[:, pl.ds(0, _CHB)],
                out_hbm.at[s, :, pl.ds(b0, _CHB)],
                sem_o[b],
            ).wait()

        def transpose_and_store(k, b):
            rv, ov = rows_v[b], outt_v[b]

            @pl.loop(0, _CHB, step=4)
            def _(j0):
                for u in range(4):
                    j = j0 + u
                    col = jnp.full((16,), j, jnp.int32)
                    for h in range(D // 16):
                        v = rv[j, pl.ds(16 * h, 16)]
                        plsc.store_scatter(ov, [rows16[h], col], v)

            s, b0 = chunk_pos(k)
            pltpu.async_copy(
                ov.at[:, pl.ds(0, _CHB)],
                out_hbm.at[s, :, pl.ds(b0, _CHB)],
                sem_o[b],
            )

        launch(0, 0)
        launch(1, 1)
        drain_gather(0)
        transpose_and_store(0, 0)
        drain_gather(1)
        launch(2, 0)
        transpose_and_store(1, 1)

        @pl.loop(2, per_w - 2, step=2)
        def _(k0):
            for b in range(2):
                k = k0 + b
                drain_gather(b)
                launch(k + 1, 1 - b)
                drain_out(k - 2, b)
                transpose_and_store(k, b)

        drain_gather(0)
        launch(per_w - 1, 1)
        drain_out(per_w - 4, 0)
        transpose_and_store(per_w - 2, 0)
        drain_gather(1)
        drain_out(per_w - 3, 1)
        transpose_and_store(per_w - 1, 1)
        drain_out(per_w - 2, 0)
        drain_out(per_w - 1, 1)

    w_rm = _tc_transpose(weight.T)
    out = gather_kernel(w_rm, tids_t)
    return out.transpose(2, 0, 1)


# speedup vs baseline: 1.2301x; 1.2301x over previous
import dataclasses
import functools

import jax
import jax.numpy as jnp
from jax import lax
from jax.experimental import pallas as pl
from jax.experimental.pallas import tpu as pltpu
from jax.experimental.pallas import tpu_sc as plsc

_NUM_CORES = 2
_NUM_SUBCORES = 16
_NUM_WORKERS = _NUM_CORES * _NUM_SUBCORES
_CHB = 512


def kernel(token_ids, weight):
    B, S = token_ids.shape
    D = weight.shape[1]
    n_chunks = (B // _CHB) * S
    per_w = n_chunks // _NUM_WORKERS
    assert B % _CHB == 0 and n_chunks % _NUM_WORKERS == 0 and per_w % 2 == 0
    blocks_per_s = B // _CHB

    tids_flat = token_ids.T.reshape(1, S * B)
    mesh = plsc.VectorSubcoreMesh(core_axis_name="c", subcore_axis_name="s")

    @functools.partial(
        pl.kernel,
        mesh=mesh,
        out_type=jax.ShapeDtypeStruct((S, D, B), weight.dtype),
        compiler_params=dataclasses.replace(
            pltpu.CompilerParams(use_tc_tiling_on_sc=False),
            needs_layout_passes=False,
        ),
        scratch_types=[
            pltpu.VMEM((1, per_w * _CHB), jnp.int32),
            pltpu.VMEM((_CHB, D), jnp.float32),
            pltpu.VMEM((_CHB, D), jnp.float32),
            pltpu.VMEM((D, _CHB + 1), jnp.float32),
            pltpu.VMEM((D, _CHB + 1), jnp.float32),
            pltpu.SemaphoreType.DMA,
            pltpu.SemaphoreType.DMA,
            pltpu.SemaphoreType.DMA,
            pltpu.SemaphoreType.DMA,
        ],
    )
    def gather_kernel(
        w_hbm, idx_hbm, out_hbm,
        idx_all, rows0, rows1, outt0, outt1, sg0, sg1, so0, so1,
    ):
        wid = lax.axis_index("s") * _NUM_CORES + lax.axis_index("c")
        base = wid * per_w
        rows_v = (rows0, rows1)
        outt_v = (outt0, outt1)
        sem_g = (sg0, sg1)
        sem_o = (so0, so1)
        rows16 = [lax.iota(jnp.int32, 16) + 16 * h for h in range(D // 16)]

        pltpu.sync_copy(
            idx_hbm.at[pl.ds(0, 1), pl.ds(base * _CHB, per_w * _CHB)], idx_all
        )

        def chunk_pos(k):
            g = base + k
            return g // blocks_per_s, (g % blocks_per_s) * _CHB

        def chunk_idx(k):
            return idx_all.at[0, pl.ds(k * _CHB, _CHB)]

        def launch(k, b):
            pltpu.async_copy(w_hbm.at[chunk_idx(k)], rows_v[b], sem_g[b])

        def drain_gather(k, b):
            pltpu.make_async_copy(
                w_hbm.at[chunk_idx(k)], rows_v[b], sem_g[b]
            ).wait()

        def drain_out(k, b):
            s, b0 = chunk_pos(k)
            pltpu.make_async_copy(
                outt_v[b].at[:, pl.ds(0, _CHB)],
                out_hbm.at[s, :, pl.ds(b0, _CHB)],
                sem_o[b],
            ).wait()

        def transpose_and_store(k, b):
            rv, ov = rows_v[b], outt_v[b]

            @pl.loop(0, _CHB, step=8)
            def _(j0):
                for u in range(8):
                    j = j0 + u
                    col = jnp.full((16,), j, jnp.int32)
                    for h in range(D // 16):
                        v = rv[j, pl.ds(16 * h, 16)]
                        plsc.store_scatter(ov, [rows16[h], col], v)

            s, b0 = chunk_pos(k)
            pltpu.async_copy(
                ov.at[:, pl.ds(0, _CHB)],
                out_hbm.at[s, :, pl.ds(b0, _CHB)],
                sem_o[b],
            )

        launch(0, 0)
        launch(1, 1)
        drain_gather(0, 0)
        transpose_and_store(0, 0)
        drain_gather(1, 1)
        launch(2, 0)
        transpose_and_store(1, 1)

        @pl.loop(2, per_w - 2, step=2)
        def _(k0):
            for b in range(2):
                k = k0 + b
                drain_gather(k, b)
                launch(k + 1, 1 - b)
                drain_out(k - 2, b)
                transpose_and_store(k, b)

        drain_gather(per_w - 2, 0)
        launch(per_w - 1, 1)
        drain_out(per_w - 4, 0)
        transpose_and_store(per_w - 2, 0)
        drain_gather(per_w - 1, 1)
        drain_out(per_w - 3, 1)
        transpose_and_store(per_w - 1, 1)
        drain_out(per_w - 2, 0)
        drain_out(per_w - 1, 1)

    out = gather_kernel(weight, tids_flat)
    return out.transpose(2, 0, 1)
